# Initial kernel scaffold; baseline (speedup 1.0000x reference)
import jax
import jax.numpy as jnp
from jax.experimental import pallas as pl


def kernel(inputs, img_feat0, img_feat1, img_feat2, img_feat3, cameras):
    raise NotImplementedError("write your pallas kernel here")


# speedup vs baseline: 1.9939x; 1.9939x over previous
import functools

import jax
import jax.numpy as jnp
import numpy as np
from jax import lax
from jax.experimental import pallas as pl
from jax.experimental.pallas import tpu as pltpu
from jax.experimental.pallas import tpu_sc as plsc

N_VIEWS = 3
SCALES = (56, 28, 14, 7)
CHANS = (64, 128, 256, 512)
COFF = (0, 64, 192, 448)
CTOT = 960
OUTW = 3 + 3 * CTOT

NC = 2
NS = 16
NW = NC * NS
LANES = 16


def _cam(param):
    theta = param[0] * (np.pi / 180.0)
    camy = param[3] * jnp.sin(param[1] * (np.pi / 180.0))
    lens = param[3] * jnp.cos(param[1] * (np.pi / 180.0))
    camx = lens * jnp.cos(theta)
    camz = lens * jnp.sin(theta)
    Z = jnp.stack([camx, camy, camz])
    x = camy * jnp.cos(theta + np.pi)
    z = camy * jnp.sin(theta + np.pi)
    Y = jnp.stack([x, lens, z])
    X = jnp.cross(Y, Z)
    cm = jnp.stack([
        X / jnp.linalg.norm(X),
        Y / jnp.linalg.norm(Y),
        Z / jnp.linalg.norm(Z),
    ])
    return cm, Z


def _fast_sqrt(x):
    i = lax.bitcast_convert_type(x, jnp.int32)
    i = jnp.int32(0x5F3759DF) - (i >> 1)
    y = lax.bitcast_convert_type(i, jnp.float32)
    for _ in range(3):
        y = y * (1.5 - 0.5 * x * y * y)
    return x * y


def _sc_kernel(n_pts, n_chunks):
    q, r = divmod(n_chunks, NW)
    max_chunks = q + (1 if r else 0)
    max_pts = max_chunks * LANES

    mesh = plsc.VectorSubcoreMesh(core_axis_name="c", subcore_axis_name="s",
                                  num_cores=NC, num_subcores=NS)

    @functools.partial(
        pl.kernel,
        out_type=jax.ShapeDtypeStruct((n_pts, OUTW), jnp.float32),
        mesh=mesh,
        compiler_params=pltpu.CompilerParams(
            use_tc_tiling_on_sc=False, needs_layout_passes=False),
        scratch_types=[
            pltpu.VMEM((max_pts,), jnp.float32),
            pltpu.VMEM((max_pts,), jnp.float32),
            pltpu.VMEM((max_pts,), jnp.float32),
            pltpu.VMEM((N_VIEWS, 3, max_pts), jnp.float32),
            pltpu.VMEM((12, LANES), jnp.int32),
            pltpu.VMEM((N_VIEWS, LANES, CHANS[0]), jnp.float32),
            pltpu.VMEM((N_VIEWS, LANES, CHANS[1]), jnp.float32),
            pltpu.VMEM((N_VIEWS, LANES, CHANS[2]), jnp.float32),
            pltpu.VMEM((N_VIEWS, LANES, CHANS[3]), jnp.float32),
            pltpu.VMEM((LANES, OUTW), jnp.float32),
            pltpu.SemaphoreType.DMA,
        ],
    )
    def body(xs_h, ys_h, zs_h, pc_h, t0_h, t1_h, t2_h, t3_h, out_h,
             xs_v, ys_v, zs_v, pc_v, idx_v, g0, g1, g2, g3,
             outb, sem):
        gbufs = (g0, g1, g2, g3)
        tabs = (t0_h, t1_h, t2_h, t3_h)

        wid = lax.axis_index("s") * NC + lax.axis_index("c")
        start_chunk = wid * q + jnp.minimum(wid, r)
        cnt = q + jnp.where(wid < r, 1, 0).astype(jnp.int32)
        p0 = start_chunk * LANES

        pltpu.sync_copy(xs_h.at[pl.ds(p0, max_pts)], xs_v)
        pltpu.sync_copy(ys_h.at[pl.ds(p0, max_pts)], ys_v)
        pltpu.sync_copy(zs_h.at[pl.ds(p0, max_pts)], zs_v)
        for i in range(N_VIEWS):
            for c in range(3):
                pltpu.sync_copy(pc_h.at[i, c, pl.ds(p0, max_pts)],
                                pc_v.at[i, c])

        lane = lax.iota(jnp.int32, LANES)
        zero = jnp.zeros((LANES,), jnp.float32)

        def chunk_body(k, _):
            base = k * LANES
            row0 = p0 + base
            xv = xs_v[pl.ds(base, LANES)]
            yv = ys_v[pl.ds(base, LANES)]
            zv = zs_v[pl.ds(base, LANES)]

            plsc.store_scatter(outb, [lane, jnp.zeros((LANES,), jnp.int32)], xv)
            plsc.store_scatter(outb, [lane, jnp.full((LANES,), 1, jnp.int32)], yv)
            plsc.store_scatter(outb, [lane, jnp.full((LANES,), 2, jnp.int32)], zv)

            for i in range(N_VIEWS):
                Xc = pc_v[i, 0, pl.ds(base, LANES)]
                Yc = pc_v[i, 1, pl.ds(base, LANES)]
                Zc = pc_v[i, 2, pl.ds(base, LANES)]
                nz = -Zc
                h = 248.0 * ((-Yc) / nz) + 112.0
                w = 248.0 * (Xc / nz) + 112.0
                h = jnp.where(h != h, zero, h)
                w = jnp.where(w != w, zero, w)
                h = jnp.minimum(jnp.maximum(h, 0.0), 223.0)
                w = jnp.minimum(jnp.maximum(w, 0.0), 223.0)
                for j in range(4):
                    inv_f = float(SCALES[j]) / 224.0
                    hi = (h * inv_f).astype(jnp.int32)
                    wi = (w * inv_f).astype(jnp.int32)
                    idx_v[i * 4 + j, :] = hi * SCALES[j] + wi

            copies = []
            for i in range(N_VIEWS):
                for j in range(4):
                    copies.append(pltpu.async_copy(
                        tabs[j].at[idx_v.at[i * 4 + j]], gbufs[j].at[i], sem))
            for cp in copies:
                cp.wait()

            def point_body(p, _):
                rows_p = p + jnp.zeros((LANES,), jnp.int32)
                for j in range(4):
                    g = gbufs[j]
                    coff = COFF[j]

                    def cvec_body(c, _):
                        col = c * LANES
                        v0 = g[0, p, pl.ds(col, LANES)]
                        v1 = g[1, p, pl.ds(col, LANES)]
                        v2 = g[2, p, pl.ds(col, LANES)]
                        mx = jnp.maximum(jnp.maximum(v0, v1), v2)
                        mn = ((v0 + v1) + v2) / 3.0
                        d0 = v0 - mn
                        d1 = v1 - mn
                        d2 = v2 - mn
                        var = ((d0 * d0 + d1 * d1) + d2 * d2) / 3.0 + 1e-12
                        sd = _fast_sqrt(var)
                        cols = lane + (3 + coff + col)
                        plsc.store_scatter(outb, [rows_p, cols], mx)
                        plsc.store_scatter(outb, [rows_p, cols + CTOT], mn)
                        plsc.store_scatter(outb, [rows_p, cols + 2 * CTOT], sd)
                        return ()

                    lax.fori_loop(0, CHANS[j] // LANES, cvec_body, ())
                return ()

            lax.fori_loop(0, LANES, point_body, ())

            pltpu.sync_copy(outb, out_h.at[pl.ds(row0, LANES)])
            return ()

        lax.fori_loop(0, cnt, chunk_body, ())

    return body


def kernel(inputs, img_feat0, img_feat1, img_feat2, img_feat3, cameras):
    n_pts = inputs.shape[0]
    n_chunks = n_pts // LANES
    assert n_pts % LANES == 0

    c0m, o0 = _cam(cameras[0])
    point_origin = inputs @ jnp.linalg.inv(c0m).T + o0
    pcs = []
    for i in range(N_VIEWS):
        cm, oi = _cam(cameras[i])
        pcs.append((point_origin - oi) @ cm.T)
    pc = jnp.transpose(jnp.stack(pcs), (0, 2, 1))

    feats = (img_feat0, img_feat1, img_feat2, img_feat3)
    tabs = [
        jnp.transpose(f[0], (1, 2, 0)).reshape(SCALES[j] * SCALES[j], CHANS[j])
        for j, f in enumerate(feats)
    ]

    q, r = divmod(n_chunks, NW)
    max_pts = (q + (1 if r else 0)) * LANES
    pad = NW * max_pts - n_pts
    xs = jnp.concatenate([inputs[:, 0], jnp.zeros((pad,), jnp.float32)])
    ys = jnp.concatenate([inputs[:, 1], jnp.zeros((pad,), jnp.float32)])
    zs = jnp.concatenate([inputs[:, 2], jnp.zeros((pad,), jnp.float32)])
    pc = jnp.concatenate(
        [pc, jnp.full((N_VIEWS, 3, pad), -1.0, jnp.float32)], axis=2)

    fn = _sc_kernel(n_pts, n_chunks)
    return fn(xs, ys, zs, pc, tabs[0], tabs[1], tabs[2], tabs[3])
